# Initial kernel scaffold; baseline (speedup 1.0000x reference)
import jax
import jax.numpy as jnp
from jax.experimental import pallas as pl


def kernel(user, pos_item, neg_item, adj_rows, adj_cols, adj_vals, sub1_rows, sub1_cols, sub1_vals, sub2_rows, sub2_cols, sub2_vals, user_embed, item_embed):
    raise NotImplementedError("write your pallas kernel here")


# speedup vs baseline: 6.5973x; 6.5973x over previous
import functools

import jax
import jax.numpy as jnp
from jax import lax
from jax.experimental import pallas as pl
from jax.experimental.pallas import tpu as pltpu
from jax.experimental.pallas import tpu_sc as plsc

_N_USER = 50000
_N_ITEM = 50000
_N = _N_USER + _N_ITEM
_D = 32
_H = 16
_B = 4096
_LAYERS = 3
_E = 1600000
_E_SUB = 1280000

_NC = 2
_NS = 16
_CHUNK = 128
_BLK = 8
_EDGE_BLK = _CHUNK * _BLK

_EP_ADJ = 1605632
_NB_ADJ = 98
_EP_SUB = 1294336
_NB_SUB = 79

_RPT = _N // _NS
_SLAB = 250


def _emit_spmm(src_ref, src_base, rows2, cols2, vals1, nblocks, s,
               acc, rows_blk, cols_blk, vals_blk, gidx, gbuf, gsem, ssem):
    ncpt = nblocks * _BLK
    chunk0 = s * ncpt

    def blk_body(b, carry):
        crow = chunk0 + b * _BLK
        pltpu.sync_copy(rows2.at[pl.ds(crow, _BLK), :], rows_blk)
        pltpu.sync_copy(cols2.at[pl.ds(crow, _BLK), :], cols_blk)
        pltpu.sync_copy(vals1.at[pl.ds(crow * _CHUNK, _EDGE_BLK)], vals_blk)

        def fire_gather(j, half):
            gi = gidx[half]
            for k in range(8):
                cv = cols_blk[j, pl.ds(k * 16, 16)]
                gi[pl.ds(k * 16, 16)] = cv + src_base
            return pltpu.async_copy(src_ref.at[gi], gbuf[half], gsem)

        def scale(j, half):
            buf = gbuf[half]

            def grp(g, c2):
                for k in range(8):
                    p = g * 8 + k
                    vi = j * _CHUNK + p
                    v = plsc.load_gather(
                        vals_blk, [jnp.full((16,), vi, jnp.int32)])
                    buf[p, :] = buf[p, :] * v
                return c2

            lax.fori_loop(0, 16, grp, 0)

        def pair(jj, c2):
            j0 = 2 * jj
            j1 = j0 + 1
            d0 = fire_gather(j0, 0)
            d1 = fire_gather(j1, 1)
            d0.wait()
            scale(j0, 0)
            s0 = pltpu.async_copy(gbuf[0], acc.at[rows_blk.at[j0]], ssem,
                                  add=True)
            d1.wait()
            scale(j1, 1)
            s1 = pltpu.async_copy(gbuf[1], acc.at[rows_blk.at[j1]], ssem,
                                  add=True)
            s0.wait()
            s1.wait()
            return c2

        lax.fori_loop(0, _BLK // 2, pair, 0)
        return carry

    lax.fori_loop(0, nblocks, blk_body, 0)


def _emit_flush(dst_ref, dst_base, s, acc, zslab):
    plsc.subcore_barrier()
    r0 = s * _RPT
    for k in range(_RPT // _SLAB):
        rr = r0 + k * _SLAB
        pltpu.sync_copy(acc.at[pl.ds(rr, _SLAB), :],
                        dst_ref.at[pl.ds(dst_base + rr, _SLAB), :])
        pltpu.sync_copy(zslab, acc.at[pl.ds(rr, _SLAB), :])
    plsc.subcore_barrier()


def _emit_outgather(idx_ref, add_off, tables, out_ref, out_base, s,
                    oidx, gidx, tbufs, gsem):
    def chunk_body(ch, carry):
        base = s * (_B // _NS) + ch * _CHUNK
        pltpu.sync_copy(idx_ref.at[pl.ds(base, _CHUNK)], oidx)
        for t, (tref, tbase) in enumerate(tables):
            for k in range(8):
                iv = oidx[pl.ds(k * 16, 16)]
                gidx[pl.ds(k * 16, 16)] = iv + (tbase + add_off)
            pltpu.async_copy(tref.at[gidx], tbufs[t], gsem).wait()

        t0, t1, t2, t3 = tbufs

        def sgrp(g, c2):
            for k in range(8):
                p = g * 8 + k
                t0[p, :] = (t0[p, :] + t1[p, :] + t2[p, :] + t3[p, :]) * 0.25
            return c2

        lax.fori_loop(0, 16, sgrp, 0)
        pltpu.sync_copy(t0, out_ref.at[pl.ds(out_base + base, _CHUNK), :])
        return carry

    lax.fori_loop(0, _B // _NS // _CHUNK, chunk_body, 0)


def _body(emb, adjr, adjc, adjv, s1r, s1c, s1v, s2r, s2c, s2v,
          user, pos, neg,
          cur_all, e1_all, e2_all,
          o_u, o_p, o_n, o_u1, o_p1, o_n1, o_u2, o_p2, o_n2,
          acc, rows_blk, cols_blk, vals_blk, gidx0, gidx1,
          gbuf0, gbuf1, t1, t2, t3, zslab, gsem, ssem):
    c = lax.axis_index("c")
    s = lax.axis_index("s")
    coff = c * _N
    n2 = 2 * _N

    r0 = coff + s * _RPT
    for k in range(_RPT // _SLAB):
        rr = r0 + k * _SLAB
        pltpu.sync_copy(emb.at[pl.ds(rr, _SLAB), :], zslab)
        pltpu.sync_copy(zslab, cur_all.at[pl.ds(rr, _SLAB), :])

    def zb(i, c2):
        zslab[i, :] = jnp.zeros((16,), jnp.float32)
        return c2

    lax.fori_loop(0, _SLAB, zb, 0)
    for k in range(_RPT // _SLAB):
        rr = s * _RPT + k * _SLAB
        pltpu.sync_copy(zslab, acc.at[pl.ds(rr, _SLAB), :])
    plsc.subcore_barrier()

    gidx = [gidx0, gidx1]
    gbuf = [gbuf0, gbuf1]
    spmm_scratch = (acc, rows_blk, cols_blk, vals_blk, gidx, gbuf, gsem, ssem)

    def layer(l, carry):
        src_l = l * n2 + coff
        src_l1 = (l + 1) * n2 + coff
        sub_dst = l * n2 + coff
        _emit_spmm(cur_all, src_l, adjr, adjc, adjv, _NB_ADJ, s,
                   *spmm_scratch)
        _emit_flush(cur_all, src_l1, s, acc, zslab)
        _emit_spmm(cur_all, src_l1, s1r, s1c, s1v, _NB_SUB, s,
                   *spmm_scratch)
        _emit_flush(e1_all, sub_dst, s, acc, zslab)
        _emit_spmm(cur_all, src_l1, s2r, s2c, s2v, _NB_SUB, s,
                   *spmm_scratch)
        _emit_flush(e2_all, sub_dst, s, acc, zslab)
        return carry

    lax.fori_loop(0, _LAYERS, layer, 0)

    cur_tabs = [(cur_all, t * n2 + coff) for t in range(4)]
    e1_tabs = [(cur_all, coff)] + [(e1_all, t * n2 + coff) for t in range(3)]
    e2_tabs = [(cur_all, coff)] + [(e2_all, t * n2 + coff) for t in range(3)]
    tbufs = [gbuf0, t1, t2, t3]
    ob = c * _B
    for idx_ref, add_off, tabs, out_ref in (
            (user, 0, cur_tabs, o_u),
            (pos, _N_USER, cur_tabs, o_p),
            (neg, _N_USER, cur_tabs, o_n),
            (user, 0, e1_tabs, o_u1),
            (pos, _N_USER, e1_tabs, o_p1),
            (neg, _N_USER, e1_tabs, o_n1),
            (user, 0, e2_tabs, o_u2),
            (pos, _N_USER, e2_tabs, o_p2),
            (neg, _N_USER, e2_tabs, o_n2)):
        _emit_outgather(idx_ref, add_off, tabs, out_ref, ob, s,
                        gidx0, gidx1, tbufs, gsem)


def _pad_edges(rows, cols, vals, ep):
    e = rows.shape[0]
    padn = ep - e
    fill = (jnp.arange(padn, dtype=jnp.int32) % _N).astype(jnp.int32)
    rows = jnp.concatenate([rows.astype(jnp.int32), fill])
    cols = jnp.concatenate([cols.astype(jnp.int32), fill])
    vals = jnp.concatenate([vals, jnp.zeros((padn,), jnp.float32)])
    return rows.reshape(-1, _CHUNK), cols.reshape(-1, _CHUNK), vals


@jax.jit
def kernel(user, pos_item, neg_item, adj_rows, adj_cols, adj_vals,
           sub1_rows, sub1_cols, sub1_vals, sub2_rows, sub2_cols, sub2_vals,
           user_embed, item_embed):
    emb = jnp.concatenate([
        jnp.concatenate([user_embed[:, :_H], item_embed[:, :_H]], axis=0),
        jnp.concatenate([user_embed[:, _H:], item_embed[:, _H:]], axis=0),
    ], axis=0)
    adjr, adjc, adjv = _pad_edges(adj_rows, adj_cols, adj_vals, _EP_ADJ)
    s1r, s1c, s1v = _pad_edges(sub1_rows, sub1_cols, sub1_vals, _EP_SUB)
    s2r, s2c, s2v = _pad_edges(sub2_rows, sub2_cols, sub2_vals, _EP_SUB)

    f32 = jnp.float32
    n2 = 2 * _N
    out_type = (
        jax.ShapeDtypeStruct(((_LAYERS + 1) * n2, _H), f32),
        jax.ShapeDtypeStruct((_LAYERS * n2, _H), f32),
        jax.ShapeDtypeStruct((_LAYERS * n2, _H), f32),
    ) + tuple(jax.ShapeDtypeStruct((2 * _B, _H), f32) for _ in range(9))

    scratch = [
        pltpu.VMEM_SHARED((_N, _H), f32),
        pltpu.VMEM((_BLK, _CHUNK), jnp.int32),
        pltpu.VMEM((_BLK, _CHUNK), jnp.int32),
        pltpu.VMEM((_EDGE_BLK,), f32),
        pltpu.VMEM((_CHUNK,), jnp.int32),
        pltpu.VMEM((_CHUNK,), jnp.int32),
        pltpu.VMEM((_CHUNK, _H), f32),
        pltpu.VMEM((_CHUNK, _H), f32),
        pltpu.VMEM((_CHUNK, _H), f32),
        pltpu.VMEM((_CHUNK, _H), f32),
        pltpu.VMEM((_CHUNK, _H), f32),
        pltpu.VMEM((_SLAB, _H), f32),
        pltpu.SemaphoreType.DMA,
        pltpu.SemaphoreType.DMA,
    ]

    mesh = plsc.VectorSubcoreMesh(core_axis_name="c", subcore_axis_name="s",
                                  num_cores=_NC, num_subcores=_NS)
    outs = pl.kernel(_body, out_type=out_type, mesh=mesh,
                     scratch_types=scratch,
                     compiler_params=pltpu.CompilerParams(
                         use_tc_tiling_on_sc=False,
                         needs_layout_passes=False))(
        emb, adjr, adjc, adjv, s1r, s1c, s1v, s2r, s2c, s2v,
        user.astype(jnp.int32), pos_item.astype(jnp.int32),
        neg_item.astype(jnp.int32))

    o = outs[3:]

    def halves(x):
        return jnp.concatenate([x[:_B], x[_B:]], axis=-1)

    u, p, n, u1, p1, n1, u2, p2, n2o = (halves(x) for x in o)
    it1 = jnp.concatenate([p1, n1], axis=0)
    it2 = jnp.concatenate([p2, n2o], axis=0)
    return (u, p, n, u1, it1, u2, it2)


# speedup vs baseline: 7.7007x; 1.1672x over previous
import jax
import jax.numpy as jnp
from jax import lax
from jax.experimental import pallas as pl
from jax.experimental.pallas import tpu as pltpu
from jax.experimental.pallas import tpu_sc as plsc

_N_USER = 50000
_N_ITEM = 50000
_N = _N_USER + _N_ITEM
_D = 32
_H = 16
_B = 4096
_LAYERS = 3
_NC = 2
_NS = 16
_CHUNK = 128
_BLK = 8
_EDGE_BLK = _CHUNK * _BLK

_EP_ADJ = 1605632
_NB_ADJ = 98
_EP_SUB = 1310720
_NB_SUB = 80

_RPT = _N // _NS
_SLAB = 250
_NSL = _RPT // _SLAB


def _emit_spmm(src_ref, src_base, rows2, cols2, vals1, nblocks, chunk0, scr):
    acc, esets, gidx, gbuf, dummy, esem, gsem, ssem = scr

    def fire_edges(bidx, si):
        r, c, v = esets[si]
        crow = chunk0 + bidx * _BLK
        pltpu.async_copy(rows2.at[pl.ds(crow, _BLK), :], r, esem)
        pltpu.async_copy(cols2.at[pl.ds(crow, _BLK), :], c, esem)
        pltpu.async_copy(vals1.at[pl.ds(crow * _CHUNK, _EDGE_BLK)], v, esem)

    def wait_edges(si):
        r, c, v = esets[si]
        pltpu.make_async_copy(rows2.at[pl.ds(0, _BLK), :], r, esem).wait()
        pltpu.make_async_copy(cols2.at[pl.ds(0, _BLK), :], c, esem).wait()
        pltpu.make_async_copy(vals1.at[pl.ds(0, _EDGE_BLK)], v, esem).wait()

    def build_fire_gather(cols_s, row, rp):
        gi = gidx[rp]
        for k in range(8):
            cv = cols_s[row, pl.ds(k * 16, 16)]
            gi[pl.ds(k * 16, 16)] = cv + src_base
        return pltpu.async_copy(src_ref.at[gi], gbuf[rp], gsem)

    def scale(j, buf, vals_s):
        def grp(g, c2):
            for k in range(8):
                p = g * 8 + k
                vi = j * _CHUNK + p
                v = plsc.load_gather(
                    vals_s, [jnp.full((16,), vi, jnp.int32)])
                buf[p, :] = buf[p, :] * v
            return c2

        lax.fori_loop(0, 16, grp, 0)

    def block_body(b, si):
        rows_s, cols_s, vals_s = esets[si]
        wait_edges(si)
        fire_edges(jnp.minimum(b + 1, nblocks - 1), 1 - si)

        def pair(jj, c2):
            j0 = 2 * jj
            j1 = j0 + 1
            d0 = build_fire_gather(cols_s, j0, 0)
            d1 = build_fire_gather(cols_s, j1, 1)
            d0.wait()
            scale(j0, gbuf[0], vals_s)
            s0 = pltpu.async_copy(gbuf[0], acc.at[rows_s.at[j0]], ssem,
                                  add=True)
            d1.wait()
            scale(j1, gbuf[1], vals_s)
            s1 = pltpu.async_copy(gbuf[1], acc.at[rows_s.at[j1]], ssem,
                                  add=True)
            s0.wait()
            s1.wait()
            return c2

        lax.fori_loop(0, _BLK // 2, pair, 0)

    fire_edges(0, 0)

    def pair_body(bb, c2):
        b0 = 2 * bb
        block_body(b0, 0)
        block_body(b0 + 1, 1)
        return c2

    lax.fori_loop(0, nblocks // 2, pair_body, 0)

    wait_edges(0)


def _emit_flush(dst_ref, dst_base, s, acc, zslab, fsem, zsem):
    plsc.subcore_barrier()
    r0 = s * _RPT

    def fire_out(k, c2):
        rr = r0 + k * _SLAB
        pltpu.async_copy(acc.at[pl.ds(rr, _SLAB), :],
                         dst_ref.at[pl.ds(dst_base + rr, _SLAB), :], fsem)
        return c2

    def wait_out(k, c2):
        pltpu.make_async_copy(acc.at[pl.ds(r0, _SLAB), :],
                              dst_ref.at[pl.ds(dst_base, _SLAB), :],
                              fsem).wait()
        return c2

    def fire_zero(k, c2):
        rr = r0 + k * _SLAB
        pltpu.async_copy(zslab, acc.at[pl.ds(rr, _SLAB), :], zsem)
        return c2

    def wait_zero(k, c2):
        pltpu.make_async_copy(zslab, acc.at[pl.ds(r0, _SLAB), :],
                              zsem).wait()
        return c2

    lax.fori_loop(0, 4, fire_out, 0)
    def out_ring(k, c2):
        wait_out(k, c2)
        return fire_out(k + 4, c2)
    lax.fori_loop(0, _NSL - 4, out_ring, 0)
    lax.fori_loop(0, 4, wait_out, 0)
    lax.fori_loop(0, 4, fire_zero, 0)
    def zero_ring(k, c2):
        wait_zero(k, c2)
        return fire_zero(k + 4, c2)
    lax.fori_loop(0, _NSL - 4, zero_ring, 0)
    lax.fori_loop(0, 4, wait_zero, 0)
    plsc.subcore_barrier()


def _emit_outgather(idx_ref, add_off, tables, out_ref, out_base, s,
                    oidx, gidx, gbuf, gsem):
    def chunk_body(ch, carry):
        base = s * (_B // _NS) + ch * _CHUNK
        pltpu.sync_copy(idx_ref.at[pl.ds(base, _CHUNK)], oidx)
        for t, (tref, tbase) in enumerate(tables):
            gi = gidx[t]
            for k in range(8):
                iv = oidx[pl.ds(k * 16, 16)]
                gi[pl.ds(k * 16, 16)] = iv + (tbase + add_off)
            pltpu.async_copy(tref.at[gi], gbuf[t], gsem)
        for t, (tref, tbase) in enumerate(tables):
            pltpu.make_async_copy(tref.at[gidx[t]], gbuf[t], gsem).wait()
        t0, t1, t2, t3 = gbuf

        def sgrp(g, c2):
            for k in range(8):
                p = g * 8 + k
                t0[p, :] = (t0[p, :] + t1[p, :] + t2[p, :] + t3[p, :]) * 0.25
            return c2

        lax.fori_loop(0, 16, sgrp, 0)
        pltpu.sync_copy(t0, out_ref.at[pl.ds(out_base + base, _CHUNK), :])
        return carry

    lax.fori_loop(0, _B // _NS // _CHUNK, chunk_body, 0)


def _body(emb, rows2, cols2, vals1,
          user, pos, neg,
          tabs,
          o_u, o_p, o_n, o_u1, o_p1, o_n1, o_u2, o_p2, o_n2,
          acc, rA, cA, vA, rB, cB, vB,
          gidx0, gidx1, gidx2, gidx3,
          gbuf0, gbuf1, gbuf2, gbuf3, dummy, oidx, zslab,
          esem, gsem, ssem, fsem, zsem):
    c = lax.axis_index("c")
    s = lax.axis_index("s")
    coff = c * _N
    n2 = 2 * _N

    esets = [(rA, cA, vA), (rB, cB, vB)]
    gidx = [gidx0, gidx1, gidx2, gidx3]
    gbuf = [gbuf0, gbuf1, gbuf2, gbuf3]

    r0 = coff + s * _RPT
    for k in range(_NSL):
        rr = r0 + k * _SLAB
        pltpu.sync_copy(emb.at[pl.ds(rr, _SLAB), :], zslab)
        pltpu.sync_copy(zslab, tabs.at[pl.ds(rr, _SLAB), :])

    def zb(i, c2):
        zslab[i, :] = jnp.zeros((16,), jnp.float32)
        return c2

    lax.fori_loop(0, _SLAB, zb, 0)

    def zfire(k, c2):
        rr = s * _RPT + k * _SLAB
        pltpu.async_copy(zslab, acc.at[pl.ds(rr, _SLAB), :], zsem)
        return c2

    def zwait(k, c2):
        pltpu.make_async_copy(zslab, acc.at[pl.ds(s * _RPT, _SLAB), :],
                              zsem).wait()
        return c2

    lax.fori_loop(0, 4, zfire, 0)

    def zring(k, c2):
        zwait(k, c2)
        return zfire(k + 4, c2)

    lax.fori_loop(0, _NSL - 4, zring, 0)
    lax.fori_loop(0, 4, zwait, 0)
    plsc.subcore_barrier()

    spmm_scr = (acc, esets, gidx, gbuf, dummy, esem, gsem, ssem)

    def step_body(step, carry):
        l = step // 3
        p = step - 3 * l
        is_adj = p == 0
        src_slot = jnp.where(is_adj, l, l + 1)
        dst_slot = jnp.where(is_adj, l + 1,
                             jnp.where(p == 1, 4 + l, 7 + l))
        nb = jnp.where(is_adj, _NB_ADJ, _NB_SUB)
        ncpt = nb * _BLK
        ebase = jnp.where(is_adj, 0,
                          jnp.where(p == 1, _EP_ADJ // _CHUNK,
                                    (_EP_ADJ + _EP_SUB) // _CHUNK))
        chunk0 = ebase + s * ncpt
        _emit_spmm(tabs, src_slot * n2 + coff, rows2, cols2, vals1,
                   nb, chunk0, spmm_scr)
        _emit_flush(tabs, dst_slot * n2 + coff, s, acc, zslab, fsem, zsem)
        return carry

    lax.fori_loop(0, 3 * _LAYERS, step_body, 0)

    cur_tabs = [(tabs, t * n2 + coff) for t in range(4)]
    e1_tabs = [(tabs, coff)] + [(tabs, (4 + t) * n2 + coff)
                                for t in range(3)]
    e2_tabs = [(tabs, coff)] + [(tabs, (7 + t) * n2 + coff)
                                for t in range(3)]
    ob = c * _B
    for idx_ref, add_off, tabs, out_ref in (
            (user, 0, cur_tabs, o_u),
            (pos, _N_USER, cur_tabs, o_p),
            (neg, _N_USER, cur_tabs, o_n),
            (user, 0, e1_tabs, o_u1),
            (pos, _N_USER, e1_tabs, o_p1),
            (neg, _N_USER, e1_tabs, o_n1),
            (user, 0, e2_tabs, o_u2),
            (pos, _N_USER, e2_tabs, o_p2),
            (neg, _N_USER, e2_tabs, o_n2)):
        _emit_outgather(idx_ref, add_off, tabs, out_ref, ob, s,
                        oidx, gidx, gbuf, gsem)


def _pad_edges(rows, cols, vals, ep):
    e = rows.shape[0]
    padn = ep - e
    fill = (jnp.arange(padn, dtype=jnp.int32) % _N).astype(jnp.int32)
    rows = jnp.concatenate([rows.astype(jnp.int32), fill])
    cols = jnp.concatenate([cols.astype(jnp.int32), fill])
    vals = jnp.concatenate([vals, jnp.zeros((padn,), jnp.float32)])
    return rows.reshape(-1, _CHUNK), cols.reshape(-1, _CHUNK), vals


@jax.jit
def kernel(user, pos_item, neg_item, adj_rows, adj_cols, adj_vals,
           sub1_rows, sub1_cols, sub1_vals, sub2_rows, sub2_cols, sub2_vals,
           user_embed, item_embed):
    emb = jnp.concatenate([
        jnp.concatenate([user_embed[:, :_H], item_embed[:, :_H]], axis=0),
        jnp.concatenate([user_embed[:, _H:], item_embed[:, _H:]], axis=0),
    ], axis=0)
    adjr, adjc, adjv = _pad_edges(adj_rows, adj_cols, adj_vals, _EP_ADJ)
    s1r, s1c, s1v = _pad_edges(sub1_rows, sub1_cols, sub1_vals, _EP_SUB)
    s2r, s2c, s2v = _pad_edges(sub2_rows, sub2_cols, sub2_vals, _EP_SUB)
    rows2 = jnp.concatenate([adjr, s1r, s2r], axis=0)
    cols2 = jnp.concatenate([adjc, s1c, s2c], axis=0)
    vals1 = jnp.concatenate([adjv, s1v, s2v], axis=0)

    f32 = jnp.float32
    n2 = 2 * _N
    out_type = (
        jax.ShapeDtypeStruct((10 * n2, _H), f32),
    ) + tuple(jax.ShapeDtypeStruct((2 * _B, _H), f32) for _ in range(9))

    scratch = [
        pltpu.VMEM_SHARED((_N, _H), f32),
        pltpu.VMEM((_BLK, _CHUNK), jnp.int32),
        pltpu.VMEM((_BLK, _CHUNK), jnp.int32),
        pltpu.VMEM((_EDGE_BLK,), f32),
        pltpu.VMEM((_BLK, _CHUNK), jnp.int32),
        pltpu.VMEM((_BLK, _CHUNK), jnp.int32),
        pltpu.VMEM((_EDGE_BLK,), f32),
        pltpu.VMEM((_CHUNK,), jnp.int32),
        pltpu.VMEM((_CHUNK,), jnp.int32),
        pltpu.VMEM((_CHUNK,), jnp.int32),
        pltpu.VMEM((_CHUNK,), jnp.int32),
        pltpu.VMEM((_CHUNK, _H), f32),
        pltpu.VMEM((_CHUNK, _H), f32),
        pltpu.VMEM((_CHUNK, _H), f32),
        pltpu.VMEM((_CHUNK, _H), f32),
        pltpu.VMEM((_CHUNK, _H), f32),
        pltpu.VMEM((_CHUNK,), jnp.int32),
        pltpu.VMEM((_SLAB, _H), f32),
        pltpu.SemaphoreType.DMA,
        pltpu.SemaphoreType.DMA,
        pltpu.SemaphoreType.DMA,
        pltpu.SemaphoreType.DMA,
        pltpu.SemaphoreType.DMA,
    ]

    mesh = plsc.VectorSubcoreMesh(core_axis_name="c", subcore_axis_name="s",
                                  num_cores=_NC, num_subcores=_NS)
    outs = pl.kernel(_body, out_type=out_type, mesh=mesh,
                     scratch_types=scratch,
                     compiler_params=pltpu.CompilerParams(
                         use_tc_tiling_on_sc=False,
                         needs_layout_passes=False))(
        emb, rows2, cols2, vals1,
        user.astype(jnp.int32), pos_item.astype(jnp.int32),
        neg_item.astype(jnp.int32))

    o = outs[1:]

    def halves(x):
        return jnp.concatenate([x[:_B], x[_B:]], axis=-1)

    u, p, n, u1, p1, n1, u2, p2, n2o = (halves(x) for x in o)
    it1 = jnp.concatenate([p1, n1], axis=0)
    it2 = jnp.concatenate([p2, n2o], axis=0)
    return (u, p, n, u1, it1, u2, it2)


# speedup vs baseline: 9.8768x; 1.2826x over previous
import jax
import jax.numpy as jnp
from jax import lax
from jax.experimental import pallas as pl
from jax.experimental.pallas import tpu as pltpu
from jax.experimental.pallas import tpu_sc as plsc

_N_USER = 50000
_N_ITEM = 50000
_N = _N_USER + _N_ITEM
_D = 32
_H = 16
_B = 4096
_LAYERS = 3
_NC = 2
_NS = 16
_CHUNK = 128
_BLK = 8
_EDGE_BLK = _CHUNK * _BLK

_EP_ADJ = 1605632
_NB_ADJ = 98
_EP_SUB = 1310720
_NB_SUB = 80

_RPT = _N // _NS
_SLAB = 250
_NSL = _RPT // _SLAB


def _emit_spmm(src_ref, src_base, rows2, cols2, vals1, nblocks, chunk0, scr):
    acc, esets, gidx, gbuf, zslab, iot, esem, gsems, ssems = scr

    def fire_edges(bidx, si):
        r, c, v = esets[si]
        crow = chunk0 + bidx * _BLK
        pltpu.async_copy(rows2.at[pl.ds(crow, _BLK), :], r, esem)
        pltpu.async_copy(cols2.at[pl.ds(crow, _BLK), :], c, esem)
        pltpu.async_copy(vals1.at[pl.ds(crow * _CHUNK, _EDGE_BLK)], v, esem)

    def wait_edges(si):
        r, c, v = esets[si]
        pltpu.make_async_copy(rows2.at[pl.ds(0, _BLK), :], r, esem).wait()
        pltpu.make_async_copy(cols2.at[pl.ds(0, _BLK), :], c, esem).wait()
        pltpu.make_async_copy(vals1.at[pl.ds(0, _EDGE_BLK)], v, esem).wait()

    def build_fire_gather(cols_s, row, rp):
        gi = gidx[rp]
        for k in range(8):
            cv = cols_s[row, pl.ds(k * 16, 16)]
            gi[pl.ds(k * 16, 16)] = cv + src_base
        pltpu.async_copy(src_ref.at[gi], gbuf[rp], gsems[rp])

    def wait_gather(rp):
        pltpu.make_async_copy(src_ref.at[gidx[rp]], gbuf[rp],
                              gsems[rp]).wait()

    def fire_scatter(rows_s, j, rp):
        pltpu.async_copy(gbuf[rp], acc.at[rows_s.at[j]], ssems[rp],
                         add=True)

    def wait_scatter(rows_s, rp):
        pltpu.make_async_copy(gbuf[rp], acc.at[rows_s.at[0]],
                              ssems[rp]).wait()

    def scale(j, buf, vals_s):
        def grp(g, c2):
            for k in range(16):
                p = g * 16 + k
                vi = j * _CHUNK + p
                v = plsc.load_gather(
                    vals_s, [jnp.full((16,), vi, jnp.int32)])
                buf[p, :] = buf[p, :] * v
            return c2

        lax.fori_loop(0, 8, grp, 0)

    def block_body(b, si):
        rows_s, cols_s, vals_s = esets[si]
        rows_o, cols_o, _ = esets[1 - si]
        for j in range(_BLK):
            wait_gather(j)
            scale(j, gbuf[j], vals_s)
            fire_scatter(rows_s, j, j)
            wait_scatter(rows_s, (j + 5) % _BLK)
            if j == 2:
                fire_edges(jnp.minimum(b + 1, nblocks - 1), 1 - si)
            if j == 4:
                wait_edges(1 - si)
            if j < 4:
                build_fire_gather(cols_s, j + 4, (j + 4) % _BLK)
            else:
                build_fire_gather(cols_o, j - 4, (j + 4) % _BLK)

    fire_edges(0, 0)
    wait_edges(0)
    for rp in (5, 6, 7):
        pltpu.async_copy(zslab.at[pl.ds(0, _CHUNK), :], acc.at[iot],
                         ssems[rp], add=True)
    for j in range(4):
        build_fire_gather(esets[0][1], j, j)

    def pair_body(bb, c2):
        b0 = 2 * bb
        block_body(b0, 0)
        block_body(b0 + 1, 1)
        return c2

    lax.fori_loop(0, nblocks // 2, pair_body, 0)

    for rp in (5, 6, 7):
        wait_scatter(esets[0][0], rp)
    for rp in range(4):
        wait_gather(rp)


def _emit_flush(dst_ref, dst_base, s, acc, zslab, fsem, zsem):
    plsc.subcore_barrier()
    r0 = s * _RPT

    def fire_out(k, c2):
        rr = r0 + k * _SLAB
        pltpu.async_copy(acc.at[pl.ds(rr, _SLAB), :],
                         dst_ref.at[pl.ds(dst_base + rr, _SLAB), :], fsem)
        return c2

    def wait_out(k, c2):
        pltpu.make_async_copy(acc.at[pl.ds(r0, _SLAB), :],
                              dst_ref.at[pl.ds(dst_base, _SLAB), :],
                              fsem).wait()
        return c2

    def fire_zero(k, c2):
        rr = r0 + k * _SLAB
        pltpu.async_copy(zslab, acc.at[pl.ds(rr, _SLAB), :], zsem)
        return c2

    def wait_zero(k, c2):
        pltpu.make_async_copy(zslab, acc.at[pl.ds(r0, _SLAB), :],
                              zsem).wait()
        return c2

    lax.fori_loop(0, 4, fire_out, 0)
    def out_ring(k, c2):
        wait_out(k, c2)
        return fire_out(k + 4, c2)
    lax.fori_loop(0, _NSL - 4, out_ring, 0)
    lax.fori_loop(0, 4, wait_out, 0)
    lax.fori_loop(0, 4, fire_zero, 0)
    def zero_ring(k, c2):
        wait_zero(k, c2)
        return fire_zero(k + 4, c2)
    lax.fori_loop(0, _NSL - 4, zero_ring, 0)
    lax.fori_loop(0, 4, wait_zero, 0)
    plsc.subcore_barrier()


def _emit_outgather(idx_ref, add_off, tables, out_ref, out_base, s,
                    oidx, gidx, gbuf, gsems):
    def chunk_body(ch, carry):
        base = s * (_B // _NS) + ch * _CHUNK
        pltpu.sync_copy(idx_ref.at[pl.ds(base, _CHUNK)], oidx)
        for t, (tref, tbase) in enumerate(tables):
            gi = gidx[t]
            for k in range(8):
                iv = oidx[pl.ds(k * 16, 16)]
                gi[pl.ds(k * 16, 16)] = iv + (tbase + add_off)
            pltpu.async_copy(tref.at[gi], gbuf[t], gsems[t])
        for t, (tref, tbase) in enumerate(tables):
            pltpu.make_async_copy(tref.at[gidx[t]], gbuf[t], gsems[t]).wait()
        t0, t1, t2, t3 = gbuf[:4]

        def sgrp(g, c2):
            for k in range(8):
                p = g * 8 + k
                t0[p, :] = (t0[p, :] + t1[p, :] + t2[p, :] + t3[p, :]) * 0.25
            return c2

        lax.fori_loop(0, 16, sgrp, 0)
        pltpu.sync_copy(t0, out_ref.at[pl.ds(out_base + base, _CHUNK), :])
        return carry

    lax.fori_loop(0, _B // _NS // _CHUNK, chunk_body, 0)


def _body(emb, rows2, cols2, vals1,
          user, pos, neg,
          tabs,
          o_u, o_p, o_n, o_u1, o_p1, o_n1, o_u2, o_p2, o_n2,
          acc, rA, cA, vA, rB, cB, vB,
          gidx0, gidx1, gidx2, gidx3, gidx4, gidx5, gidx6, gidx7,
          gbuf0, gbuf1, gbuf2, gbuf3, gbuf4, gbuf5, gbuf6, gbuf7,
          iot, oidx, zslab,
          esem, gs0, gs1, gs2, gs3, gs4, gs5, gs6, gs7,
          ss0, ss1, ss2, ss3, ss4, ss5, ss6, ss7, fsem, zsem):
    c = lax.axis_index("c")
    s = lax.axis_index("s")
    coff = c * _N
    n2 = 2 * _N

    esets = [(rA, cA, vA), (rB, cB, vB)]
    gidx = [gidx0, gidx1, gidx2, gidx3, gidx4, gidx5, gidx6, gidx7]
    gbuf = [gbuf0, gbuf1, gbuf2, gbuf3, gbuf4, gbuf5, gbuf6, gbuf7]
    gsems = [gs0, gs1, gs2, gs3, gs4, gs5, gs6, gs7]
    ssems = [ss0, ss1, ss2, ss3, ss4, ss5, ss6, ss7]

    for k in range(8):
        iot[pl.ds(k * 16, 16)] = jnp.arange(16, dtype=jnp.int32) + (k * 16)

    r0 = coff + s * _RPT
    for k in range(_NSL):
        rr = r0 + k * _SLAB
        pltpu.sync_copy(emb.at[pl.ds(rr, _SLAB), :], zslab)
        pltpu.sync_copy(zslab, tabs.at[pl.ds(rr, _SLAB), :])

    def zb(i, c2):
        zslab[i, :] = jnp.zeros((16,), jnp.float32)
        return c2

    lax.fori_loop(0, _SLAB, zb, 0)

    def zfire(k, c2):
        rr = s * _RPT + k * _SLAB
        pltpu.async_copy(zslab, acc.at[pl.ds(rr, _SLAB), :], zsem)
        return c2

    def zwait(k, c2):
        pltpu.make_async_copy(zslab, acc.at[pl.ds(s * _RPT, _SLAB), :],
                              zsem).wait()
        return c2

    lax.fori_loop(0, 4, zfire, 0)

    def zring(k, c2):
        zwait(k, c2)
        return zfire(k + 4, c2)

    lax.fori_loop(0, _NSL - 4, zring, 0)
    lax.fori_loop(0, 4, zwait, 0)
    plsc.subcore_barrier()

    spmm_scr = (acc, esets, gidx, gbuf, zslab, iot, esem, gsems, ssems)

    def step_body(step, carry):
        l = step // 3
        p = step - 3 * l
        is_adj = p == 0
        src_slot = jnp.where(is_adj, l, l + 1)
        dst_slot = jnp.where(is_adj, l + 1,
                             jnp.where(p == 1, 4 + l, 7 + l))
        nb = jnp.where(is_adj, _NB_ADJ, _NB_SUB)
        ncpt = nb * _BLK
        ebase = jnp.where(is_adj, 0,
                          jnp.where(p == 1, _EP_ADJ // _CHUNK,
                                    (_EP_ADJ + _EP_SUB) // _CHUNK))
        chunk0 = ebase + s * ncpt
        _emit_spmm(tabs, src_slot * n2 + coff, rows2, cols2, vals1,
                   nb, chunk0, spmm_scr)
        _emit_flush(tabs, dst_slot * n2 + coff, s, acc, zslab, fsem, zsem)
        return carry

    lax.fori_loop(0, 3 * _LAYERS, step_body, 0)

    cur_tabs = [(tabs, t * n2 + coff) for t in range(4)]
    e1_tabs = [(tabs, coff)] + [(tabs, (4 + t) * n2 + coff)
                                for t in range(3)]
    e2_tabs = [(tabs, coff)] + [(tabs, (7 + t) * n2 + coff)
                                for t in range(3)]
    ob = c * _B
    for idx_ref, add_off, tabs, out_ref in (
            (user, 0, cur_tabs, o_u),
            (pos, _N_USER, cur_tabs, o_p),
            (neg, _N_USER, cur_tabs, o_n),
            (user, 0, e1_tabs, o_u1),
            (pos, _N_USER, e1_tabs, o_p1),
            (neg, _N_USER, e1_tabs, o_n1),
            (user, 0, e2_tabs, o_u2),
            (pos, _N_USER, e2_tabs, o_p2),
            (neg, _N_USER, e2_tabs, o_n2)):
        _emit_outgather(idx_ref, add_off, tabs, out_ref, ob, s,
                        oidx, gidx, gbuf, gsems)


def _pad_edges(rows, cols, vals, ep):
    e = rows.shape[0]
    padn = ep - e
    fill = (jnp.arange(padn, dtype=jnp.int32) % _N).astype(jnp.int32)
    rows = jnp.concatenate([rows.astype(jnp.int32), fill])
    cols = jnp.concatenate([cols.astype(jnp.int32), fill])
    vals = jnp.concatenate([vals, jnp.zeros((padn,), jnp.float32)])
    return rows.reshape(-1, _CHUNK), cols.reshape(-1, _CHUNK), vals


@jax.jit
def kernel(user, pos_item, neg_item, adj_rows, adj_cols, adj_vals,
           sub1_rows, sub1_cols, sub1_vals, sub2_rows, sub2_cols, sub2_vals,
           user_embed, item_embed):
    emb = jnp.concatenate([
        jnp.concatenate([user_embed[:, :_H], item_embed[:, :_H]], axis=0),
        jnp.concatenate([user_embed[:, _H:], item_embed[:, _H:]], axis=0),
    ], axis=0)
    adjr, adjc, adjv = _pad_edges(adj_rows, adj_cols, adj_vals, _EP_ADJ)
    s1r, s1c, s1v = _pad_edges(sub1_rows, sub1_cols, sub1_vals, _EP_SUB)
    s2r, s2c, s2v = _pad_edges(sub2_rows, sub2_cols, sub2_vals, _EP_SUB)
    rows2 = jnp.concatenate([adjr, s1r, s2r], axis=0)
    cols2 = jnp.concatenate([adjc, s1c, s2c], axis=0)
    vals1 = jnp.concatenate([adjv, s1v, s2v], axis=0)

    f32 = jnp.float32
    n2 = 2 * _N
    out_type = (
        jax.ShapeDtypeStruct((10 * n2, _H), f32),
    ) + tuple(jax.ShapeDtypeStruct((2 * _B, _H), f32) for _ in range(9))

    scratch = [
        pltpu.VMEM_SHARED((_N, _H), f32),
        pltpu.VMEM((_BLK, _CHUNK), jnp.int32),
        pltpu.VMEM((_BLK, _CHUNK), jnp.int32),
        pltpu.VMEM((_EDGE_BLK,), f32),
        pltpu.VMEM((_BLK, _CHUNK), jnp.int32),
        pltpu.VMEM((_BLK, _CHUNK), jnp.int32),
        pltpu.VMEM((_EDGE_BLK,), f32),
    ] + [pltpu.VMEM((_CHUNK,), jnp.int32) for _ in range(8)
    ] + [pltpu.VMEM((_CHUNK, _H), f32) for _ in range(8)
    ] + [
        pltpu.VMEM((_CHUNK,), jnp.int32),
        pltpu.VMEM((_CHUNK,), jnp.int32),
        pltpu.VMEM((_SLAB, _H), f32),
    ] + [pltpu.SemaphoreType.DMA for _ in range(19)]

    mesh = plsc.VectorSubcoreMesh(core_axis_name="c", subcore_axis_name="s",
                                  num_cores=_NC, num_subcores=_NS)
    outs = pl.kernel(_body, out_type=out_type, mesh=mesh,
                     scratch_types=scratch,
                     compiler_params=pltpu.CompilerParams(
                         use_tc_tiling_on_sc=False,
                         needs_layout_passes=False))(
        emb, rows2, cols2, vals1,
        user.astype(jnp.int32), pos_item.astype(jnp.int32),
        neg_item.astype(jnp.int32))

    o = outs[1:]

    def halves(x):
        return jnp.concatenate([x[:_B], x[_B:]], axis=-1)

    u, p, n, u1, p1, n1, u2, p2, n2o = (halves(x) for x in o)
    it1 = jnp.concatenate([p1, n1], axis=0)
    it2 = jnp.concatenate([p2, n2o], axis=0)
    return (u, p, n, u1, it1, u2, it2)


# speedup vs baseline: 28.0349x; 2.8385x over previous
import jax
import jax.numpy as jnp
from jax import lax
from jax.experimental import pallas as pl
from jax.experimental.pallas import tpu as pltpu
from jax.experimental.pallas import tpu_sc as plsc

_N_USER = 50000
_N_ITEM = 50000
_N = _N_USER + _N_ITEM
_D = 32
_H = 16
_B = 4096
_LAYERS = 3
_NC = 2
_NS = 16
_CHUNK = 128
_BLK = 8
_EDGE_BLK = _CHUNK * _BLK

_EP_ADJ = 1605632
_NB_ADJ = 98
_EP_SUB = 1310720
_NB_SUB = 80

_RPT = _N // _NS
_SLAB = 250
_NSL = _RPT // _SLAB

_DNUMS = jax.lax.GatherDimensionNumbers(
    offset_dims=(), collapsed_slice_dims=(0,), start_index_map=(0,))


def _emit_spmm(src_ref, src_base, rows2, cols2, vals1, nblocks, chunk0, scr):
    acc, esets, gidx, gbuf, zslab, iot, esem, gsems, ssems = scr

    def fire_edges(bidx, si):
        r, c, v = esets[si]
        crow = chunk0 + bidx * _BLK
        pltpu.async_copy(rows2.at[pl.ds(crow, _BLK), :], r, esem)
        pltpu.async_copy(cols2.at[pl.ds(crow, _BLK), :], c, esem)
        pltpu.async_copy(vals1.at[pl.ds(crow * _CHUNK, _EDGE_BLK)], v, esem)

    def wait_edges(si):
        r, c, v = esets[si]
        pltpu.make_async_copy(rows2.at[pl.ds(0, _BLK), :], r, esem).wait()
        pltpu.make_async_copy(cols2.at[pl.ds(0, _BLK), :], c, esem).wait()
        pltpu.make_async_copy(vals1.at[pl.ds(0, _EDGE_BLK)], v, esem).wait()

    def build_fire_gather(cols_s, row, rp):
        gi = gidx[rp]
        for k in range(8):
            cv = cols_s[row, pl.ds(k * 16, 16)]
            gi[pl.ds(k * 16, 16)] = cv + src_base
        pltpu.async_copy(src_ref.at[gi], gbuf[rp], gsems[rp])

    def wait_gather(rp):
        pltpu.make_async_copy(src_ref.at[gidx[rp]], gbuf[rp],
                              gsems[rp]).wait()

    def fire_scatter(rows_s, j, rp):
        pltpu.async_copy(gbuf[rp], acc.at[rows_s.at[j]], ssems[rp],
                         add=True)

    def wait_scatter(rows_s, rp):
        pltpu.make_async_copy(gbuf[rp], acc.at[rows_s.at[0]],
                              ssems[rp]).wait()

    def scale(j, buf, vals_s):
        def grp(g, c2):
            v16 = vals_s[pl.ds(j * _CHUNK + g * 16, 16)]
            for k in range(16):
                p = g * 16 + k
                bc = lax.gather(v16, jnp.full((16, 1), k, jnp.int32),
                                _DNUMS, (1,),
                                mode=lax.GatherScatterMode.PROMISE_IN_BOUNDS)
                buf[p, :] = buf[p, :] * bc
            return c2

        lax.fori_loop(0, 8, grp, 0)

    def block_body(b, si):
        rows_s, cols_s, vals_s = esets[si]
        rows_o, cols_o, _ = esets[1 - si]
        for j in range(_BLK):
            wait_gather(j)
            scale(j, gbuf[j], vals_s)
            fire_scatter(rows_s, j, j)
            wait_scatter(rows_s, (j + 5) % _BLK)
            if j == 2:
                fire_edges(jnp.minimum(b + 1, nblocks - 1), 1 - si)
            if j == 4:
                wait_edges(1 - si)
            if j < 4:
                build_fire_gather(cols_s, j + 4, (j + 4) % _BLK)
            else:
                build_fire_gather(cols_o, j - 4, (j + 4) % _BLK)

    fire_edges(0, 0)
    wait_edges(0)
    for rp in (5, 6, 7):
        pltpu.async_copy(zslab.at[pl.ds(0, _CHUNK), :], acc.at[iot],
                         ssems[rp], add=True)
    for j in range(4):
        build_fire_gather(esets[0][1], j, j)

    def pair_body(bb, c2):
        b0 = 2 * bb
        block_body(b0, 0)
        block_body(b0 + 1, 1)
        return c2

    lax.fori_loop(0, nblocks // 2, pair_body, 0)

    for rp in (5, 6, 7):
        wait_scatter(esets[0][0], rp)
    for rp in range(4):
        wait_gather(rp)


def _emit_flush(dst_ref, dst_base, s, acc, zslab, fsem, zsem):
    plsc.subcore_barrier()
    r0 = s * _RPT

    def fire_out(k, c2):
        rr = r0 + k * _SLAB
        pltpu.async_copy(acc.at[pl.ds(rr, _SLAB), :],
                         dst_ref.at[pl.ds(dst_base + rr, _SLAB), :], fsem)
        return c2

    def wait_out(k, c2):
        pltpu.make_async_copy(acc.at[pl.ds(r0, _SLAB), :],
                              dst_ref.at[pl.ds(dst_base, _SLAB), :],
                              fsem).wait()
        return c2

    def fire_zero(k, c2):
        rr = r0 + k * _SLAB
        pltpu.async_copy(zslab, acc.at[pl.ds(rr, _SLAB), :], zsem)
        return c2

    def wait_zero(k, c2):
        pltpu.make_async_copy(zslab, acc.at[pl.ds(r0, _SLAB), :],
                              zsem).wait()
        return c2

    lax.fori_loop(0, 4, fire_out, 0)
    def out_ring(k, c2):
        wait_out(k, c2)
        return fire_out(k + 4, c2)
    lax.fori_loop(0, _NSL - 4, out_ring, 0)
    lax.fori_loop(0, 4, wait_out, 0)
    lax.fori_loop(0, 4, fire_zero, 0)
    def zero_ring(k, c2):
        wait_zero(k, c2)
        return fire_zero(k + 4, c2)
    lax.fori_loop(0, _NSL - 4, zero_ring, 0)
    lax.fori_loop(0, 4, wait_zero, 0)
    plsc.subcore_barrier()


def _emit_outgather(idx_ref, add_off, tables, out_ref, out_base, s,
                    oidx, gidx, gbuf, gsems):
    def chunk_body(ch, carry):
        base = s * (_B // _NS) + ch * _CHUNK
        pltpu.sync_copy(idx_ref.at[pl.ds(base, _CHUNK)], oidx)
        for t, (tref, tbase) in enumerate(tables):
            gi = gidx[t]
            for k in range(8):
                iv = oidx[pl.ds(k * 16, 16)]
                gi[pl.ds(k * 16, 16)] = iv + (tbase + add_off)
            pltpu.async_copy(tref.at[gi], gbuf[t], gsems[t])
        for t, (tref, tbase) in enumerate(tables):
            pltpu.make_async_copy(tref.at[gidx[t]], gbuf[t], gsems[t]).wait()
        t0, t1, t2, t3 = gbuf[:4]

        def sgrp(g, c2):
            for k in range(8):
                p = g * 8 + k
                t0[p, :] = (t0[p, :] + t1[p, :] + t2[p, :] + t3[p, :]) * 0.25
            return c2

        lax.fori_loop(0, 16, sgrp, 0)
        pltpu.sync_copy(t0, out_ref.at[pl.ds(out_base + base, _CHUNK), :])
        return carry

    lax.fori_loop(0, _B // _NS // _CHUNK, chunk_body, 0)


def _body(emb, rows2, cols2, vals1,
          user, pos, neg,
          tabs,
          o_u, o_p, o_n, o_u1, o_p1, o_n1, o_u2, o_p2, o_n2,
          acc, rA, cA, vA, rB, cB, vB,
          gidx0, gidx1, gidx2, gidx3, gidx4, gidx5, gidx6, gidx7,
          gbuf0, gbuf1, gbuf2, gbuf3, gbuf4, gbuf5, gbuf6, gbuf7,
          iot, oidx, zslab,
          esem, gs0, gs1, gs2, gs3, gs4, gs5, gs6, gs7,
          ss0, ss1, ss2, ss3, ss4, ss5, ss6, ss7, fsem, zsem):
    c = lax.axis_index("c")
    s = lax.axis_index("s")
    coff = c * _N
    n2 = 2 * _N

    esets = [(rA, cA, vA), (rB, cB, vB)]
    gidx = [gidx0, gidx1, gidx2, gidx3, gidx4, gidx5, gidx6, gidx7]
    gbuf = [gbuf0, gbuf1, gbuf2, gbuf3, gbuf4, gbuf5, gbuf6, gbuf7]
    gsems = [gs0, gs1, gs2, gs3, gs4, gs5, gs6, gs7]
    ssems = [ss0, ss1, ss2, ss3, ss4, ss5, ss6, ss7]

    for k in range(8):
        iot[pl.ds(k * 16, 16)] = jnp.arange(16, dtype=jnp.int32) + (k * 16)

    r0 = coff + s * _RPT
    for k in range(_NSL):
        rr = r0 + k * _SLAB
        pltpu.sync_copy(emb.at[pl.ds(rr, _SLAB), :], zslab)
        pltpu.sync_copy(zslab, tabs.at[pl.ds(rr, _SLAB), :])

    def zb(i, c2):
        zslab[i, :] = jnp.zeros((16,), jnp.float32)
        return c2

    lax.fori_loop(0, _SLAB, zb, 0)

    def zfire(k, c2):
        rr = s * _RPT + k * _SLAB
        pltpu.async_copy(zslab, acc.at[pl.ds(rr, _SLAB), :], zsem)
        return c2

    def zwait(k, c2):
        pltpu.make_async_copy(zslab, acc.at[pl.ds(s * _RPT, _SLAB), :],
                              zsem).wait()
        return c2

    lax.fori_loop(0, 4, zfire, 0)

    def zring(k, c2):
        zwait(k, c2)
        return zfire(k + 4, c2)

    lax.fori_loop(0, _NSL - 4, zring, 0)
    lax.fori_loop(0, 4, zwait, 0)
    plsc.subcore_barrier()

    spmm_scr = (acc, esets, gidx, gbuf, zslab, iot, esem, gsems, ssems)

    def step_body(step, carry):
        l = step // 3
        p = step - 3 * l
        is_adj = p == 0
        src_slot = jnp.where(is_adj, l, l + 1)
        dst_slot = jnp.where(is_adj, l + 1,
                             jnp.where(p == 1, 4 + l, 7 + l))
        nb = jnp.where(is_adj, _NB_ADJ, _NB_SUB)
        ncpt = nb * _BLK
        ebase = jnp.where(is_adj, 0,
                          jnp.where(p == 1, _EP_ADJ // _CHUNK,
                                    (_EP_ADJ + _EP_SUB) // _CHUNK))
        chunk0 = ebase + s * ncpt
        _emit_spmm(tabs, src_slot * n2 + coff, rows2, cols2, vals1,
                   nb, chunk0, spmm_scr)
        _emit_flush(tabs, dst_slot * n2 + coff, s, acc, zslab, fsem, zsem)
        return carry

    lax.fori_loop(0, 3 * _LAYERS, step_body, 0)

    cur_tabs = [(tabs, t * n2 + coff) for t in range(4)]
    e1_tabs = [(tabs, coff)] + [(tabs, (4 + t) * n2 + coff)
                                for t in range(3)]
    e2_tabs = [(tabs, coff)] + [(tabs, (7 + t) * n2 + coff)
                                for t in range(3)]
    ob = c * _B
    for idx_ref, add_off, tabs, out_ref in (
            (user, 0, cur_tabs, o_u),
            (pos, _N_USER, cur_tabs, o_p),
            (neg, _N_USER, cur_tabs, o_n),
            (user, 0, e1_tabs, o_u1),
            (pos, _N_USER, e1_tabs, o_p1),
            (neg, _N_USER, e1_tabs, o_n1),
            (user, 0, e2_tabs, o_u2),
            (pos, _N_USER, e2_tabs, o_p2),
            (neg, _N_USER, e2_tabs, o_n2)):
        _emit_outgather(idx_ref, add_off, tabs, out_ref, ob, s,
                        oidx, gidx, gbuf, gsems)


def _pad_edges(rows, cols, vals, ep):
    e = rows.shape[0]
    padn = ep - e
    fill = (jnp.arange(padn, dtype=jnp.int32) % _N).astype(jnp.int32)
    rows = jnp.concatenate([rows.astype(jnp.int32), fill])
    cols = jnp.concatenate([cols.astype(jnp.int32), fill])
    vals = jnp.concatenate([vals, jnp.zeros((padn,), jnp.float32)])
    return rows.reshape(-1, _CHUNK), cols.reshape(-1, _CHUNK), vals


@jax.jit
def kernel(user, pos_item, neg_item, adj_rows, adj_cols, adj_vals,
           sub1_rows, sub1_cols, sub1_vals, sub2_rows, sub2_cols, sub2_vals,
           user_embed, item_embed):
    emb = jnp.concatenate([
        jnp.concatenate([user_embed[:, :_H], item_embed[:, :_H]], axis=0),
        jnp.concatenate([user_embed[:, _H:], item_embed[:, _H:]], axis=0),
    ], axis=0)
    adjr, adjc, adjv = _pad_edges(adj_rows, adj_cols, adj_vals, _EP_ADJ)
    s1r, s1c, s1v = _pad_edges(sub1_rows, sub1_cols, sub1_vals, _EP_SUB)
    s2r, s2c, s2v = _pad_edges(sub2_rows, sub2_cols, sub2_vals, _EP_SUB)
    rows2 = jnp.concatenate([adjr, s1r, s2r], axis=0)
    cols2 = jnp.concatenate([adjc, s1c, s2c], axis=0)
    vals1 = jnp.concatenate([adjv, s1v, s2v], axis=0)

    f32 = jnp.float32
    n2 = 2 * _N
    out_type = (
        jax.ShapeDtypeStruct((10 * n2, _H), f32),
    ) + tuple(jax.ShapeDtypeStruct((2 * _B, _H), f32) for _ in range(9))

    scratch = [
        pltpu.VMEM_SHARED((_N, _H), f32),
        pltpu.VMEM((_BLK, _CHUNK), jnp.int32),
        pltpu.VMEM((_BLK, _CHUNK), jnp.int32),
        pltpu.VMEM((_EDGE_BLK,), f32),
        pltpu.VMEM((_BLK, _CHUNK), jnp.int32),
        pltpu.VMEM((_BLK, _CHUNK), jnp.int32),
        pltpu.VMEM((_EDGE_BLK,), f32),
    ] + [pltpu.VMEM((_CHUNK,), jnp.int32) for _ in range(8)
    ] + [pltpu.VMEM((_CHUNK, _H), f32) for _ in range(8)
    ] + [
        pltpu.VMEM((_CHUNK,), jnp.int32),
        pltpu.VMEM((_CHUNK,), jnp.int32),
        pltpu.VMEM((_SLAB, _H), f32),
    ] + [pltpu.SemaphoreType.DMA for _ in range(19)]

    mesh = plsc.VectorSubcoreMesh(core_axis_name="c", subcore_axis_name="s",
                                  num_cores=_NC, num_subcores=_NS)
    outs = pl.kernel(_body, out_type=out_type, mesh=mesh,
                     scratch_types=scratch,
                     compiler_params=pltpu.CompilerParams(
                         use_tc_tiling_on_sc=False,
                         needs_layout_passes=False))(
        emb, rows2, cols2, vals1,
        user.astype(jnp.int32), pos_item.astype(jnp.int32),
        neg_item.astype(jnp.int32))

    o = outs[1:]

    def halves(x):
        return jnp.concatenate([x[:_B], x[_B:]], axis=-1)

    u, p, n, u1, p1, n1, u2, p2, n2o = (halves(x) for x in o)
    it1 = jnp.concatenate([p1, n1], axis=0)
    it2 = jnp.concatenate([p2, n2o], axis=0)
    return (u, p, n, u1, it1, u2, it2)
